# baseline (speedup 1.0000x reference)
import functools

import jax
import jax.numpy as jnp
from jax import lax
from jax.experimental import pallas as pl
from jax.experimental.pallas import tpu as pltpu
from jax.experimental.pallas import tpu_sc as plsc

_B = 16384
_HIDDEN = 128
_ITEM_NUM = 100000
_MAX_SLIP = 0.4
_MAX_GUESS = 0.4
_T = 50.0

_NC = 2
_NS = 16
_NW = _NC * _NS
_B_PER_W = _B // _NW
_SG_ROWS = 1564


def _sc_gather(user, sg_row_idx, theta_table, sg_flat):
    mesh = plsc.VectorSubcoreMesh(core_axis_name="c", subcore_axis_name="s")

    @functools.partial(
        pl.kernel,
        out_type=(
            jax.ShapeDtypeStruct((_B, _HIDDEN), jnp.float32),
            jax.ShapeDtypeStruct((_B, _HIDDEN), jnp.float32),
        ),
        mesh=mesh,
        scratch_types=[
            pltpu.VMEM((_B_PER_W,), jnp.int32),
            pltpu.VMEM((_B_PER_W, _HIDDEN), jnp.float32),
            pltpu.VMEM((_B_PER_W,), jnp.int32),
            pltpu.VMEM((_B_PER_W // 2, _HIDDEN), jnp.float32),
            pltpu.SemaphoreType.DMA,
            pltpu.SemaphoreType.DMA,
        ],
    )
    def gather_kernel(user_hbm, sgi_hbm, theta_hbm, sg_hbm,
                      theta_out, sg_out,
                      uidx_v, rows_v, iidx_v, sg_v, sem_t, sem_s):
        wid = lax.axis_index("s") * _NC + lax.axis_index("c")
        base = wid * _B_PER_W
        pltpu.sync_copy(user_hbm.at[pl.ds(base, _B_PER_W)], uidx_v)
        pltpu.sync_copy(sgi_hbm.at[pl.ds(base, _B_PER_W)], iidx_v)
        cp_t = pltpu.async_copy(theta_hbm.at[uidx_v], rows_v, sem_t)
        half = _B_PER_W // 2

        @pl.loop(0, 2)
        def _(h):
            off = h * half
            cp_s = pltpu.async_copy(
                sg_hbm.at[iidx_v.at[pl.ds(off, half)]], sg_v, sem_s)
            cp_s.wait()
            pltpu.sync_copy(sg_v, sg_out.at[pl.ds(base + off, half)])

        cp_t.wait()
        pltpu.sync_copy(rows_v, theta_out.at[pl.ds(base, _B_PER_W)])

    return gather_kernel(user, sg_row_idx, theta_table, sg_flat)


def _score_block(theta_ref, kn_ref, sgrow_ref, lane_ref, out_ref):
    th = theta_ref[...]
    kn = kn_ref[...]
    n = jnp.sum(kn * (jax.nn.sigmoid(th) - 0.5), axis=1, keepdims=True)
    s = jax.nn.sigmoid(n * (1.0 / _T))

    sgrow = sgrow_ref[...]
    lane0 = lane_ref[...]
    lanes = lax.broadcasted_iota(jnp.int32, sgrow.shape, 1)
    slip_raw = jnp.sum(jnp.where(lanes == lane0, sgrow, 0.0), axis=1,
                       keepdims=True)
    guess_raw = jnp.sum(jnp.where(lanes == lane0 + 1, sgrow, 0.0), axis=1,
                        keepdims=True)
    slip = jax.nn.sigmoid(slip_raw) * _MAX_SLIP
    guess = jax.nn.sigmoid(guess_raw) * _MAX_GUESS
    out_ref[...] = (1.0 - slip) * s + guess * (1.0 - s)


def kernel(user, item, knowledge, theta_table, slip_table, guess_table):
    sg_flat = jnp.concatenate([slip_table, guess_table], axis=1).reshape(-1)
    sg_flat = jnp.pad(sg_flat, (0, _SG_ROWS * 128 - 2 * _ITEM_NUM))
    sg_flat = sg_flat.reshape(_SG_ROWS, 128)

    sg_row_idx = (item // 64).astype(jnp.int32)
    lane0 = ((item % 64) * 2).astype(jnp.int32).reshape(_B, 1)

    theta_g, sg_g = _sc_gather(user, sg_row_idx, theta_table, sg_flat)

    rows = 2048
    out = pl.pallas_call(
        _score_block,
        grid=(_B // rows,),
        in_specs=[
            pl.BlockSpec((rows, _HIDDEN), lambda i: (i, 0)),
            pl.BlockSpec((rows, _HIDDEN), lambda i: (i, 0)),
            pl.BlockSpec((rows, _HIDDEN), lambda i: (i, 0)),
            pl.BlockSpec((rows, 1), lambda i: (i, 0)),
        ],
        out_specs=pl.BlockSpec((rows, 1), lambda i: (i, 0)),
        out_shape=jax.ShapeDtypeStruct((_B, 1), jnp.float32),
    )(theta_g, knowledge, sg_g, lane0)
    return out.reshape(_B)
